# baseline (speedup 1.0000x reference)
import functools

import jax
import jax.numpy as jnp
from jax.experimental import pallas as pl

_E_TILE = 4096


def _align_body(p_ref, e_ref, v_ref, out_ref, cnt_ref, *, n_events):
    Tp = p_ref.shape[0]
    D = v_ref.shape[1]
    p_col = p_ref[...]
    p_iota = jax.lax.broadcasted_iota(jnp.int32, (Tp, 1), 0).astype(jnp.float32)

    out_ref[...] = jnp.zeros((Tp, D), jnp.float32)
    cnt_ref[...] = jnp.zeros((Tp, 1), jnp.float32)

    n_tiles = n_events // _E_TILE

    def step(t, _):
        e_row = e_ref[:, pl.ds(t * _E_TILE, _E_TILE)]
        dist = jnp.abs(p_col - e_row)
        min_d = jnp.min(dist, axis=0, keepdims=True)
        masked = jnp.where(dist == min_d, p_iota, jnp.float32(Tp))
        min_idx = jnp.min(masked, axis=0, keepdims=True)
        oh_t = (p_iota == min_idx).astype(jnp.float32)
        vals = v_ref[pl.ds(t * _E_TILE, _E_TILE), :]
        out_ref[...] += jnp.dot(oh_t, vals, preferred_element_type=jnp.float32)
        cnt_ref[...] += jnp.sum(oh_t, axis=1, keepdims=True)
        return 0

    jax.lax.fori_loop(0, n_tiles, step, 0)
    out_ref[...] = out_ref[...] / jnp.maximum(cnt_ref[...], 1.0)


def kernel(price_timestamps, event_timestamps, event_values):
    B, Tp = price_timestamps.shape
    Te = event_timestamps.shape[1]
    D = event_values.shape[2]

    out, counts = pl.pallas_call(
        functools.partial(_align_body, n_events=Te),
        grid=(B,),
        in_specs=[
            pl.BlockSpec((None, Tp, 1), lambda b: (b, 0, 0)),
            pl.BlockSpec((None, 1, Te), lambda b: (b, 0, 0)),
            pl.BlockSpec((None, Te, D), lambda b: (b, 0, 0)),
        ],
        out_specs=[
            pl.BlockSpec((None, Tp, D), lambda b: (b, 0, 0)),
            pl.BlockSpec((None, Tp, 1), lambda b: (b, 0, 0)),
        ],
        out_shape=[
            jax.ShapeDtypeStruct((B, Tp, D), jnp.float32),
            jax.ShapeDtypeStruct((B, Tp, 1), jnp.float32),
        ],
    )(
        price_timestamps.reshape(B, Tp, 1),
        event_timestamps.reshape(B, 1, Te),
        event_values,
    )
    return out, counts.reshape(B, Tp) > 0


# speedup vs baseline: 1.0388x; 1.0388x over previous
import functools

import jax
import jax.numpy as jnp
from jax.experimental import pallas as pl

_E_TILE = 2048


def _align_body(p_ref, e_ref, v_ref, out_ref, cnt_ref, *, n_events):
    Tp = p_ref.shape[0]
    D = v_ref.shape[1]
    p_col = p_ref[...]
    p_iota = jax.lax.broadcasted_iota(jnp.int32, (Tp, 1), 0).astype(jnp.float32)

    out_ref[...] = jnp.zeros((Tp, D), jnp.float32)
    cnt_ref[...] = jnp.zeros((Tp, 1), jnp.float32)

    n_tiles = n_events // _E_TILE

    def step(t, _):
        e_row = e_ref[:, pl.ds(t * _E_TILE, _E_TILE)]
        dist = jnp.abs(p_col - e_row)
        min_d = jnp.min(dist, axis=0, keepdims=True)
        masked = jnp.where(dist == min_d, p_iota, jnp.float32(Tp))
        min_idx = jnp.min(masked, axis=0, keepdims=True)
        oh_t = (p_iota == min_idx).astype(jnp.bfloat16)
        vals = v_ref[pl.ds(t * _E_TILE, _E_TILE), :].astype(jnp.bfloat16)
        out_ref[...] += jnp.dot(oh_t, vals, preferred_element_type=jnp.float32)
        cnt_ref[...] += jnp.sum(oh_t.astype(jnp.float32), axis=1, keepdims=True)
        return 0

    jax.lax.fori_loop(0, n_tiles, step, 0)
    out_ref[...] = out_ref[...] / jnp.maximum(cnt_ref[...], 1.0)


def kernel(price_timestamps, event_timestamps, event_values):
    B, Tp = price_timestamps.shape
    Te = event_timestamps.shape[1]
    D = event_values.shape[2]

    out, counts = pl.pallas_call(
        functools.partial(_align_body, n_events=Te),
        grid=(B,),
        in_specs=[
            pl.BlockSpec((None, Tp, 1), lambda b: (b, 0, 0)),
            pl.BlockSpec((None, 1, Te), lambda b: (b, 0, 0)),
            pl.BlockSpec((None, Te, D), lambda b: (b, 0, 0)),
        ],
        out_specs=[
            pl.BlockSpec((None, Tp, D), lambda b: (b, 0, 0)),
            pl.BlockSpec((None, Tp, 1), lambda b: (b, 0, 0)),
        ],
        out_shape=[
            jax.ShapeDtypeStruct((B, Tp, D), jnp.float32),
            jax.ShapeDtypeStruct((B, Tp, 1), jnp.float32),
        ],
    )(
        price_timestamps.reshape(B, Tp, 1),
        event_timestamps.reshape(B, 1, Te),
        event_values,
    )
    return out, counts.reshape(B, Tp) > 0


# speedup vs baseline: 1.0633x; 1.0236x over previous
import functools

import jax
import jax.numpy as jnp
from jax.experimental import pallas as pl

_E_TILE = 2048


def _align_body(p_ref, e_ref, v_ref, out_ref, cnt_ref, *, n_events):
    Tp = p_ref.shape[0]
    D = v_ref.shape[1]
    p_col = p_ref[...]
    p_iota = jax.lax.broadcasted_iota(jnp.int32, (Tp, 1), 0).astype(jnp.float32)

    out_ref[...] = jnp.zeros((Tp, D), jnp.float32)
    cnt_ref[...] = jnp.zeros((Tp, 1), jnp.float32)

    n_tiles = n_events // _E_TILE

    def step(t, _):
        e_row = e_ref[:, pl.ds(t * _E_TILE, _E_TILE)]
        dist = jnp.abs(p_col - e_row)
        min_d = jnp.min(dist, axis=0, keepdims=True)
        masked = jnp.where(dist == min_d, p_iota, jnp.float32(Tp))
        min_idx = jnp.min(masked, axis=0, keepdims=True)
        oh_t = (p_iota == min_idx).astype(jnp.float32)
        vals = v_ref[pl.ds(t * _E_TILE, _E_TILE), :]
        out_ref[...] += jnp.dot(oh_t, vals, preferred_element_type=jnp.float32)
        cnt_ref[...] += jnp.sum(oh_t, axis=1, keepdims=True)
        return 0

    jax.lax.fori_loop(0, n_tiles, step, 0)
    out_ref[...] = out_ref[...] / jnp.maximum(cnt_ref[...], 1.0)


def kernel(price_timestamps, event_timestamps, event_values):
    B, Tp = price_timestamps.shape
    Te = event_timestamps.shape[1]
    D = event_values.shape[2]

    out, counts = pl.pallas_call(
        functools.partial(_align_body, n_events=Te),
        grid=(B,),
        in_specs=[
            pl.BlockSpec((None, Tp, 1), lambda b: (b, 0, 0)),
            pl.BlockSpec((None, 1, Te), lambda b: (b, 0, 0)),
            pl.BlockSpec((None, Te, D), lambda b: (b, 0, 0)),
        ],
        out_specs=[
            pl.BlockSpec((None, Tp, D), lambda b: (b, 0, 0)),
            pl.BlockSpec((None, Tp, 1), lambda b: (b, 0, 0)),
        ],
        out_shape=[
            jax.ShapeDtypeStruct((B, Tp, D), jnp.float32),
            jax.ShapeDtypeStruct((B, Tp, 1), jnp.float32),
        ],
    )(
        price_timestamps.reshape(B, Tp, 1),
        event_timestamps.reshape(B, 1, Te),
        event_values,
    )
    return out, counts.reshape(B, Tp) > 0
